# initial kernel scaffold (unmeasured)
import jax
import jax.numpy as jnp
from jax import lax
from jax.experimental import pallas as pl
from jax.experimental.pallas import tpu as pltpu

B = 32
H = 16
D = 128
BS = 32
NP_LOCAL = 256
T = NP_LOCAL * BS
SCALE = D ** -0.5
NEG = -1e30


def _partial_body(q_ref, k_ref, v_ref, w_ref, acc_ref, m_ref, l_ref):
    q = q_ref[0]
    k = k_ref[:, 0, :]
    v = v_ref[:, 0, :]
    w = w_ref[...]
    s = lax.dot_general(
        q, k, (((1,), (1,)), ((), ())), preferred_element_type=jnp.float32
    ) * SCALE
    s = jnp.where(w > 0, s, NEG)
    m = jnp.max(s, axis=1, keepdims=True)
    e = jnp.exp(s - m) * w
    l_ref[0] = jnp.sum(e, axis=1)
    m_ref[0] = m[:, 0]
    acc_ref[0] = lax.dot_general(
        e, v, (((1,), (0,)), ((), ())), preferred_element_type=jnp.float32
    )


def _partials(q_hbd, k_thd, v_thd, w_bt):
    return pl.pallas_call(
        _partial_body,
        grid=(H,),
        in_specs=[
            pl.BlockSpec((1, B, D), lambda h: (h, 0, 0)),
            pl.BlockSpec((T, 1, D), lambda h: (0, h, 0)),
            pl.BlockSpec((T, 1, D), lambda h: (0, h, 0)),
            pl.BlockSpec((B, T), lambda h: (0, 0)),
        ],
        out_specs=[
            pl.BlockSpec((1, B, D), lambda h: (h, 0, 0)),
            pl.BlockSpec((1, B), lambda h: (h, 0)),
            pl.BlockSpec((1, B), lambda h: (h, 0)),
        ],
        out_shape=[
            jax.ShapeDtypeStruct((H, B, D), jnp.float32),
            jax.ShapeDtypeStruct((H, B), jnp.float32),
            jax.ShapeDtypeStruct((H, B), jnp.float32),
        ],
    )(q_hbd, k_thd, v_thd, w_bt)


def _combine_body(
    acc_ref, m_ref, l_ref, out_ref, cacc, cm, cl, send_sems, recv_sems
):
    my_x = lax.axis_index("x")
    my_y = lax.axis_index("y")
    peer = (my_x, 1 - my_y)

    barrier = pltpu.get_barrier_semaphore()
    pl.semaphore_signal(
        barrier, inc=1, device_id=peer, device_id_type=pl.DeviceIdType.MESH
    )
    pl.semaphore_wait(barrier, 1)

    copies = []
    for i, (src, dst) in enumerate(
        ((acc_ref, cacc), (m_ref, cm), (l_ref, cl))
    ):
        c = pltpu.make_async_remote_copy(
            src_ref=src,
            dst_ref=dst,
            send_sem=send_sems.at[i],
            recv_sem=recv_sems.at[i],
            device_id=peer,
            device_id_type=pl.DeviceIdType.MESH,
        )
        c.start()
        copies.append(c)
    for c in copies:
        c.wait()

    m0 = m_ref[...]
    m1 = cm[...]
    mn = jnp.maximum(m0, m1)
    a0 = jnp.exp(m0 - mn)
    a1 = jnp.exp(m1 - mn)
    l = l_ref[...] * a0 + cl[...] * a1
    o = acc_ref[...] * a0[..., None] + cacc[...] * a1[..., None]
    out_ref[...] = o / l[..., None]


def _combine(acc, m, l):
    return pl.pallas_call(
        _combine_body,
        out_shape=jax.ShapeDtypeStruct((H, B, D), jnp.float32),
        in_specs=[pl.BlockSpec(memory_space=pltpu.VMEM)] * 3,
        out_specs=pl.BlockSpec(memory_space=pltpu.VMEM),
        scratch_shapes=[
            pltpu.VMEM((H, B, D), jnp.float32),
            pltpu.VMEM((H, B), jnp.float32),
            pltpu.VMEM((H, B), jnp.float32),
            pltpu.SemaphoreType.DMA((3,)),
            pltpu.SemaphoreType.DMA((3,)),
        ],
        compiler_params=pltpu.CompilerParams(collective_id=0),
    )(acc, m, l)


def kernel(Q, K, V, bt, lens):
    my_y = lax.axis_index("y")
    pages = my_y * NP_LOCAL + jnp.arange(NP_LOCAL, dtype=jnp.int32)
    valid = jnp.arange(bt.shape[1], dtype=jnp.int32)[None, :] < lens[:, None]
    counts = jnp.sum(
        (bt[:, :, None] == pages[None, None, :]) & valid[:, :, None],
        axis=1,
        dtype=jnp.float32,
    )
    w_bt = jnp.repeat(counts, BS, axis=1)

    q_hbd = jnp.transpose(Q[:, 0, :, :], (1, 0, 2))
    k_thd = K.reshape(T, H, D)
    v_thd = V.reshape(T, H, D)

    acc, m, l = _partials(q_hbd, k_thd, v_thd, w_bt)
    out = _combine(acc, m, l)
    return jnp.transpose(out, (1, 0, 2))[:, None, :, :]


# baseline (device time: 160250 ns/iter reference)
import jax
import jax.numpy as jnp
from jax import lax
from jax.experimental import pallas as pl
from jax.experimental.pallas import tpu as pltpu

B = 32
H = 16
D = 128
BS = 32
NP_LOCAL = 256
T = NP_LOCAL * BS
SCALE = D ** -0.5
NEG = -1e30


def _partial_body(q_ref, k_ref, v_ref, w_ref, acc_ref, m_ref, l_ref):
    q = q_ref[0]
    k = k_ref[...]
    v = v_ref[...]
    w = w_ref[...]
    s = lax.dot_general(
        q, k, (((1,), (1,)), ((), ())), preferred_element_type=jnp.float32
    ) * SCALE
    s = jnp.where(w > 0, s, NEG)
    m = jnp.max(s, axis=1, keepdims=True)
    e = jnp.exp(s - m) * w
    l = jnp.sum(e, axis=1, keepdims=True)
    acc_ref[0] = lax.dot_general(
        e, v, (((1,), (0,)), ((), ())), preferred_element_type=jnp.float32
    )
    m_ref[0] = jnp.broadcast_to(m, (B, D))
    l_ref[0] = jnp.broadcast_to(l, (B, D))


def _partials(q_hbd, k_t_hd, v_t_hd, w_bt):
    return pl.pallas_call(
        _partial_body,
        grid=(H,),
        in_specs=[
            pl.BlockSpec((1, B, D), lambda h: (h, 0, 0)),
            pl.BlockSpec((T, D), lambda h: (0, h)),
            pl.BlockSpec((T, D), lambda h: (0, h)),
            pl.BlockSpec((B, T), lambda h: (0, 0)),
        ],
        out_specs=[
            pl.BlockSpec((1, B, D), lambda h: (h, 0, 0)),
            pl.BlockSpec((1, B, D), lambda h: (h, 0, 0)),
            pl.BlockSpec((1, B, D), lambda h: (h, 0, 0)),
        ],
        out_shape=[
            jax.ShapeDtypeStruct((H, B, D), jnp.float32),
            jax.ShapeDtypeStruct((H, B, D), jnp.float32),
            jax.ShapeDtypeStruct((H, B, D), jnp.float32),
        ],
    )(q_hbd, k_t_hd, v_t_hd, w_bt)


def _combine_body(
    acc_ref, m_ref, l_ref, out_ref, cacc, cm, cl, send_sems, recv_sems
):
    my_x = lax.axis_index("x")
    my_y = lax.axis_index("y")
    peer = (my_x, 1 - my_y)

    barrier = pltpu.get_barrier_semaphore()
    pl.semaphore_signal(
        barrier, inc=1, device_id=peer, device_id_type=pl.DeviceIdType.MESH
    )
    pl.semaphore_wait(barrier, 1)

    copies = []
    for i, (src, dst) in enumerate(
        ((acc_ref, cacc), (m_ref, cm), (l_ref, cl))
    ):
        c = pltpu.make_async_remote_copy(
            src_ref=src,
            dst_ref=dst,
            send_sem=send_sems.at[i],
            recv_sem=recv_sems.at[i],
            device_id=peer,
            device_id_type=pl.DeviceIdType.MESH,
        )
        c.start()
        copies.append(c)
    for c in copies:
        c.wait()

    m0 = m_ref[...]
    m1 = cm[...]
    mn = jnp.maximum(m0, m1)
    a0 = jnp.exp(m0 - mn)
    a1 = jnp.exp(m1 - mn)
    l = l_ref[...] * a0 + cl[...] * a1
    o = acc_ref[...] * a0 + cacc[...] * a1
    out_ref[...] = o / l


def _combine(acc, m, l):
    return pl.pallas_call(
        _combine_body,
        out_shape=jax.ShapeDtypeStruct((H, B, D), jnp.float32),
        in_specs=[pl.BlockSpec(memory_space=pltpu.VMEM)] * 3,
        out_specs=pl.BlockSpec(memory_space=pltpu.VMEM),
        scratch_shapes=[
            pltpu.VMEM((H, B, D), jnp.float32),
            pltpu.VMEM((H, B, D), jnp.float32),
            pltpu.VMEM((H, B, D), jnp.float32),
            pltpu.SemaphoreType.DMA((3,)),
            pltpu.SemaphoreType.DMA((3,)),
        ],
        compiler_params=pltpu.CompilerParams(collective_id=0),
    )(acc, m, l)


def kernel(Q, K, V, bt, lens):
    my_y = lax.axis_index("y")
    pages = my_y * NP_LOCAL + jnp.arange(NP_LOCAL, dtype=jnp.int32)
    valid = jnp.arange(bt.shape[1], dtype=jnp.int32)[None, :] < lens[:, None]
    counts = jnp.sum(
        (bt[:, :, None] == pages[None, None, :]) & valid[:, :, None],
        axis=1,
        dtype=jnp.float32,
    )
    w_bt = jnp.repeat(counts, BS, axis=1)

    q_hbd = jnp.transpose(Q[:, 0, :, :], (1, 0, 2))
    k_t_hd = K.reshape(T, H * D)
    v_t_hd = V.reshape(T, H * D)

    acc, m, l = _partials(q_hbd, k_t_hd, v_t_hd, w_bt)
    out = _combine(acc, m, l)
    return jnp.transpose(out, (1, 0, 2))[:, None, :, :]


# device time: 64946 ns/iter; 2.4674x vs baseline; 2.4674x over previous
import jax
import jax.numpy as jnp
from jax import lax
from jax.experimental import pallas as pl
from jax.experimental.pallas import tpu as pltpu

B = 32
H = 16
D = 128
BS = 32
NP_LOCAL = 256
T = NP_LOCAL * BS
SCALE = D ** -0.5
NEG = -1e30


def _body(
    q_ref,
    k_any,
    v_any,
    w_ref,
    out_ref,
    kbuf, vbuf,
    pacc, pm, pl_,
    cacc, cm, cl,
    ksems, vsems,
    send_sems, recv_sems,
):
    my_x = lax.axis_index("x")
    my_y = lax.axis_index("y")
    peer = (my_x, 1 - my_y)

    barrier = pltpu.get_barrier_semaphore()
    pl.semaphore_signal(
        barrier, inc=1, device_id=peer, device_id_type=pl.DeviceIdType.MESH
    )
    pl.semaphore_wait(barrier, 1)

    def issue(h, slot):
        pltpu.make_async_copy(
            k_any.at[:, h, :], kbuf.at[slot], ksems.at[slot]
        ).start()
        pltpu.make_async_copy(
            v_any.at[:, h, :], vbuf.at[slot], vsems.at[slot]
        ).start()

    w = w_ref[...]
    wmask = w > 0

    issue(0, 0)
    for h in range(H):
        slot = h % 2
        if h + 1 < H:
            issue(h + 1, (h + 1) % 2)
        pltpu.make_async_copy(
            k_any.at[:, h, :], kbuf.at[slot], ksems.at[slot]
        ).wait()
        pltpu.make_async_copy(
            v_any.at[:, h, :], vbuf.at[slot], vsems.at[slot]
        ).wait()

        q = q_ref[h]
        s = lax.dot_general(
            q, kbuf[slot],
            (((1,), (1,)), ((), ())),
            preferred_element_type=jnp.float32,
        ) * SCALE
        s = jnp.where(wmask, s, NEG)
        m = jnp.max(s, axis=1, keepdims=True)
        e = jnp.exp(s - m) * w
        l = jnp.sum(e, axis=1, keepdims=True)
        pacc[h] = lax.dot_general(
            e, vbuf[slot],
            (((1,), (0,)), ((), ())),
            preferred_element_type=jnp.float32,
        )
        pm[h] = jnp.broadcast_to(m, (B, D))
        pl_[h] = jnp.broadcast_to(l, (B, D))

    copies = []
    for i, (src, dst) in enumerate(((pacc, cacc), (pm, cm), (pl_, cl))):
        c = pltpu.make_async_remote_copy(
            src_ref=src,
            dst_ref=dst,
            send_sem=send_sems.at[i],
            recv_sem=recv_sems.at[i],
            device_id=peer,
            device_id_type=pl.DeviceIdType.MESH,
        )
        c.start()
        copies.append(c)
    for c in copies:
        c.wait()

    m0 = pm[...]
    m1 = cm[...]
    mn = jnp.maximum(m0, m1)
    a0 = jnp.exp(m0 - mn)
    a1 = jnp.exp(m1 - mn)
    lsum = pl_[...] * a0 + cl[...] * a1
    out_ref[...] = (pacc[...] * a0 + cacc[...] * a1) / lsum


def kernel(Q, K, V, bt, lens):
    my_y = lax.axis_index("y")
    pages = my_y * NP_LOCAL + jnp.arange(NP_LOCAL, dtype=jnp.int32)
    valid = jnp.arange(bt.shape[1], dtype=jnp.int32)[None, :] < lens[:, None]
    counts = jnp.sum(
        (bt[:, :, None] == pages[None, None, :]) & valid[:, :, None],
        axis=1,
        dtype=jnp.float32,
    )
    w_bt = jnp.repeat(counts, BS, axis=1)

    q_hbd = jnp.transpose(Q[:, 0, :, :], (1, 0, 2))
    k_thd = K.reshape(T, H, D)
    v_thd = V.reshape(T, H, D)

    out = pl.pallas_call(
        _body,
        out_shape=jax.ShapeDtypeStruct((H, B, D), jnp.float32),
        in_specs=[
            pl.BlockSpec(memory_space=pltpu.VMEM),
            pl.BlockSpec(memory_space=pltpu.MemorySpace.HBM),
            pl.BlockSpec(memory_space=pltpu.MemorySpace.HBM),
            pl.BlockSpec(memory_space=pltpu.VMEM),
        ],
        out_specs=pl.BlockSpec(memory_space=pltpu.VMEM),
        scratch_shapes=[
            pltpu.VMEM((2, T, D), jnp.float32),
            pltpu.VMEM((2, T, D), jnp.float32),
            pltpu.VMEM((H, B, D), jnp.float32),
            pltpu.VMEM((H, B, D), jnp.float32),
            pltpu.VMEM((H, B, D), jnp.float32),
            pltpu.VMEM((H, B, D), jnp.float32),
            pltpu.VMEM((H, B, D), jnp.float32),
            pltpu.VMEM((H, B, D), jnp.float32),
            pltpu.SemaphoreType.DMA((2,)),
            pltpu.SemaphoreType.DMA((2,)),
            pltpu.SemaphoreType.DMA((3,)),
            pltpu.SemaphoreType.DMA((3,)),
        ],
        compiler_params=pltpu.CompilerParams(collective_id=0),
    )(q_hbd, k_thd, v_thd, w_bt)

    return jnp.transpose(out, (1, 0, 2))[:, None, :, :]


# device time: 41486 ns/iter; 3.8627x vs baseline; 1.5655x over previous
import jax
import jax.numpy as jnp
from jax import lax
from jax.experimental import pallas as pl
from jax.experimental.pallas import tpu as pltpu

B = 32
H = 16
HL = H // 2
D = 128
BS = 32
NP_LOCAL = 256
T = NP_LOCAL * BS
SCALE = D ** -0.5
NEG = -1e30


def _body(
    q_ref,
    k_any,
    v_any,
    w_ref,
    out_ref,
    kbuf, vbuf,
    packed, cpacked,
    ksems, vsems,
    ysend, yrecv,
    xsend, xrecv,
):
    my_x = lax.axis_index("x")
    my_y = lax.axis_index("y")
    ypeer = (my_x, 1 - my_y)
    xpeer = (1 - my_x, my_y)
    h0 = my_x * HL

    barrier = pltpu.get_barrier_semaphore()
    for nbr in (ypeer, xpeer):
        pl.semaphore_signal(
            barrier, inc=1, device_id=nbr, device_id_type=pl.DeviceIdType.MESH
        )
    pl.semaphore_wait(barrier, 2)

    def issue(hl, slot):
        h = h0 + hl
        pltpu.make_async_copy(
            k_any.at[:, h, :], kbuf.at[slot], ksems.at[slot]
        ).start()
        pltpu.make_async_copy(
            v_any.at[:, h, :], vbuf.at[slot], vsems.at[slot]
        ).start()

    w = w_ref[...]
    wmask = w > 0

    yrdmas = []
    issue(0, 0)
    for hl in range(HL):
        slot = hl % 2
        if hl + 1 < HL:
            issue(hl + 1, (hl + 1) % 2)
        pltpu.make_async_copy(
            k_any.at[:, h0 + hl, :], kbuf.at[slot], ksems.at[slot]
        ).wait()
        pltpu.make_async_copy(
            v_any.at[:, h0 + hl, :], vbuf.at[slot], vsems.at[slot]
        ).wait()

        q = q_ref[hl]
        s = lax.dot_general(
            q, kbuf[slot],
            (((1,), (1,)), ((), ())),
            preferred_element_type=jnp.float32,
        ) * SCALE
        s = jnp.where(wmask, s, NEG)
        m = jnp.max(s, axis=1, keepdims=True)
        e = jnp.exp(s - m) * w
        l = jnp.sum(e, axis=1, keepdims=True)
        packed[hl, 0] = lax.dot_general(
            e, vbuf[slot],
            (((1,), (0,)), ((), ())),
            preferred_element_type=jnp.float32,
        )
        packed[hl, 1] = jnp.broadcast_to(m, (B, D))
        packed[hl, 2] = jnp.broadcast_to(l, (B, D))

        c = pltpu.make_async_remote_copy(
            src_ref=packed.at[hl],
            dst_ref=cpacked.at[hl],
            send_sem=ysend.at[hl],
            recv_sem=yrecv.at[hl],
            device_id=ypeer,
            device_id_type=pl.DeviceIdType.MESH,
        )
        c.start()
        yrdmas.append(c)

    for c in yrdmas:
        c.wait()

    m0 = packed[:, 1]
    m1 = cpacked[:, 1]
    mn = jnp.maximum(m0, m1)
    a0 = jnp.exp(m0 - mn)
    a1 = jnp.exp(m1 - mn)
    lsum = packed[:, 2] * a0 + cpacked[:, 2] * a1
    out_ref[pl.ds(h0, HL)] = (packed[:, 0] * a0 + cpacked[:, 0] * a1) / lsum

    xc = pltpu.make_async_remote_copy(
        src_ref=out_ref.at[pl.ds(h0, HL)],
        dst_ref=out_ref.at[pl.ds(h0, HL)],
        send_sem=xsend.at[0],
        recv_sem=xrecv.at[0],
        device_id=xpeer,
        device_id_type=pl.DeviceIdType.MESH,
    )
    xc.start()
    xc.wait()


def kernel(Q, K, V, bt, lens):
    my_x = lax.axis_index("x")
    my_y = lax.axis_index("y")
    pages = my_y * NP_LOCAL + jnp.arange(NP_LOCAL, dtype=jnp.int32)
    valid = jnp.arange(bt.shape[1], dtype=jnp.int32)[None, :] < lens[:, None]
    counts = jnp.sum(
        (bt[:, :, None] == pages[None, None, :]) & valid[:, :, None],
        axis=1,
        dtype=jnp.float32,
    )
    w_bt = jnp.repeat(counts, BS, axis=1)

    q_hbd = jnp.transpose(Q[:, 0, :, :], (1, 0, 2))
    q_half = lax.dynamic_slice_in_dim(q_hbd, my_x * HL, HL, axis=0)
    k_thd = K.reshape(T, H, D)
    v_thd = V.reshape(T, H, D)

    out = pl.pallas_call(
        _body,
        out_shape=jax.ShapeDtypeStruct((H, B, D), jnp.float32),
        in_specs=[
            pl.BlockSpec(memory_space=pltpu.VMEM),
            pl.BlockSpec(memory_space=pltpu.MemorySpace.HBM),
            pl.BlockSpec(memory_space=pltpu.MemorySpace.HBM),
            pl.BlockSpec(memory_space=pltpu.VMEM),
        ],
        out_specs=pl.BlockSpec(memory_space=pltpu.VMEM),
        scratch_shapes=[
            pltpu.VMEM((2, T, D), jnp.float32),
            pltpu.VMEM((2, T, D), jnp.float32),
            pltpu.VMEM((HL, 3, B, D), jnp.float32),
            pltpu.VMEM((HL, 3, B, D), jnp.float32),
            pltpu.SemaphoreType.DMA((2,)),
            pltpu.SemaphoreType.DMA((2,)),
            pltpu.SemaphoreType.DMA((HL,)),
            pltpu.SemaphoreType.DMA((HL,)),
            pltpu.SemaphoreType.DMA((1,)),
            pltpu.SemaphoreType.DMA((1,)),
        ],
        compiler_params=pltpu.CompilerParams(collective_id=0),
    )(q_half, k_thd, v_thd, w_bt)

    return jnp.transpose(out, (1, 0, 2))[:, None, :, :]


# device time: 31700 ns/iter; 5.0552x vs baseline; 1.3087x over previous
import jax
import jax.numpy as jnp
from jax import lax
from jax.experimental import pallas as pl
from jax.experimental.pallas import tpu as pltpu

B = 32
H = 16
HL = H // 2
D = 128
BS = 32
NP_LOCAL = 256
NB = 256
T = NP_LOCAL * BS
C = 2
SLOTS = 2
CH = T // C
SCALE = D ** -0.5
NEG = -1e30


def _body(
    q_ref,
    bt_ref,
    lens_ref,
    k_any,
    v_any,
    out_ref,
    kbuf, vbuf,
    packed, cpacked,
    ksems, vsems,
    ysend, yrecv,
    xsend, xrecv,
):
    my_x = lax.axis_index("x")
    my_y = lax.axis_index("y")
    ypeer = (my_x, 1 - my_y)
    xpeer = (1 - my_x, my_y)
    h0 = my_x * HL

    def issue(hl, slot):
        h = h0 + hl
        for c in range(C):
            rows = pl.ds(c * CH, CH)
            pltpu.make_async_copy(
                k_any.at[rows, h, :], kbuf.at[slot, rows], ksems.at[slot, c]
            ).start()
            pltpu.make_async_copy(
                v_any.at[rows, h, :], vbuf.at[slot, rows], vsems.at[slot, c]
            ).start()

    def wait(hl, slot):
        h = h0 + hl
        for c in range(C):
            rows = pl.ds(c * CH, CH)
            pltpu.make_async_copy(
                k_any.at[rows, h, :], kbuf.at[slot, rows], ksems.at[slot, c]
            ).wait()
            pltpu.make_async_copy(
                v_any.at[rows, h, :], vbuf.at[slot, rows], vsems.at[slot, c]
            ).wait()

    for pf in range(min(SLOTS - 1, HL)):
        issue(pf, pf % SLOTS)

    barrier = pltpu.get_barrier_semaphore()
    for nbr in (ypeer, xpeer):
        pl.semaphore_signal(
            barrier, inc=1, device_id=nbr, device_id_type=pl.DeviceIdType.MESH
        )
    pl.semaphore_wait(barrier, 2)

    jvalid = (
        lax.broadcasted_iota(jnp.int32, (B, NB), 1) < lens_ref[...]
    )
    btm = jnp.where(jvalid, bt_ref[...], -1)
    page_ids = my_y * NP_LOCAL + lax.broadcasted_iota(
        jnp.int32, (B, NB, NP_LOCAL), 2
    )
    counts = jnp.sum(
        (btm[:, :, None] == page_ids).astype(jnp.float32), axis=1
    )
    rep = (
        lax.broadcasted_iota(jnp.int32, (NP_LOCAL, T), 1) // BS
        == lax.broadcasted_iota(jnp.int32, (NP_LOCAL, T), 0)
    ).astype(jnp.bfloat16)
    logc = jnp.where(counts > 0, jnp.log(counts), NEG)
    logw = lax.dot_general(
        logc.astype(jnp.bfloat16), rep,
        (((1,), (0,)), ((), ())),
        preferred_element_type=jnp.float32,
    )
    qs_all = (q_ref[...] * SCALE).astype(jnp.bfloat16)

    yrdmas = []
    xrdmas = []

    def merge_and_push(hl):
        yrdmas[hl].wait()
        m0 = packed[hl, 1]
        m1 = cpacked[hl, 1]
        mn = jnp.maximum(m0, m1)
        a0 = jnp.exp(m0 - mn)
        a1 = jnp.exp(m1 - mn)
        lsum = packed[hl, 2] * a0 + cpacked[hl, 2] * a1
        out_ref[h0 + hl] = (packed[hl, 0] * a0 + cpacked[hl, 0] * a1) / lsum
        xc = pltpu.make_async_remote_copy(
            src_ref=out_ref.at[h0 + hl],
            dst_ref=out_ref.at[h0 + hl],
            send_sem=xsend.at[hl],
            recv_sem=xrecv.at[hl],
            device_id=xpeer,
            device_id_type=pl.DeviceIdType.MESH,
        )
        xc.start()
        xrdmas.append(xc)

    for hl in range(HL):
        slot = hl % SLOTS
        if hl + SLOTS - 1 < HL:
            issue(hl + SLOTS - 1, (hl + SLOTS - 1) % SLOTS)
        wait(hl, slot)

        s = lax.dot_general(
            qs_all[hl], kbuf[slot].astype(jnp.bfloat16),
            (((1,), (1,)), ((), ())),
            preferred_element_type=jnp.float32,
        ) + logw
        m = jnp.max(s, axis=1, keepdims=True)
        e = jnp.exp(s - m)
        l = jnp.sum(e, axis=1, keepdims=True)
        packed[hl, 0] = lax.dot_general(
            e.astype(jnp.bfloat16), vbuf[slot].astype(jnp.bfloat16),
            (((1,), (0,)), ((), ())),
            preferred_element_type=jnp.float32,
        )
        packed[hl, 1] = jnp.broadcast_to(m, (B, D))
        packed[hl, 2] = jnp.broadcast_to(l, (B, D))

        c = pltpu.make_async_remote_copy(
            src_ref=packed.at[hl],
            dst_ref=cpacked.at[hl],
            send_sem=ysend.at[hl],
            recv_sem=yrecv.at[hl],
            device_id=ypeer,
            device_id_type=pl.DeviceIdType.MESH,
        )
        c.start()
        yrdmas.append(c)

        if hl >= 1:
            merge_and_push(hl - 1)

    merge_and_push(HL - 1)
    for xc in xrdmas:
        xc.wait()


def kernel(Q, K, V, bt, lens):
    my_x = lax.axis_index("x")

    q_hbd = jnp.transpose(Q[:, 0, :, :], (1, 0, 2))
    q_half = lax.dynamic_slice_in_dim(q_hbd, my_x * HL, HL, axis=0)
    k_thd = K.reshape(T, H, D)
    v_thd = V.reshape(T, H, D)

    out = pl.pallas_call(
        _body,
        out_shape=jax.ShapeDtypeStruct((H, B, D), jnp.float32),
        in_specs=[
            pl.BlockSpec(memory_space=pltpu.VMEM),
            pl.BlockSpec(memory_space=pltpu.VMEM),
            pl.BlockSpec(memory_space=pltpu.VMEM),
            pl.BlockSpec(memory_space=pltpu.MemorySpace.HBM),
            pl.BlockSpec(memory_space=pltpu.MemorySpace.HBM),
        ],
        out_specs=pl.BlockSpec(memory_space=pltpu.VMEM),
        scratch_shapes=[
            pltpu.VMEM((SLOTS, T, D), jnp.float32),
            pltpu.VMEM((SLOTS, T, D), jnp.float32),
            pltpu.VMEM((HL, 3, B, D), jnp.float32),
            pltpu.VMEM((HL, 3, B, D), jnp.float32),
            pltpu.SemaphoreType.DMA((SLOTS, C)),
            pltpu.SemaphoreType.DMA((SLOTS, C)),
            pltpu.SemaphoreType.DMA((HL,)),
            pltpu.SemaphoreType.DMA((HL,)),
            pltpu.SemaphoreType.DMA((1,)),
            pltpu.SemaphoreType.DMA((1,)),
        ],
        compiler_params=pltpu.CompilerParams(collective_id=0),
    )(q_half, bt, lens.reshape(B, 1), k_thd, v_thd)

    return jnp.transpose(out, (1, 0, 2))[:, None, :, :]
